# baseline (speedup 1.0000x reference)
import functools

import jax
import jax.numpy as jnp
from jax import lax
from jax.experimental import pallas as pl
from jax.experimental.pallas import tpu as pltpu
from jax.experimental.pallas import tpu_sc as plsc

VOCAB = 100000
DIM = 64
ROWS = 4096 * 50
IDX_MINOR = 128
NUM_WORKERS = 32
ROWS_PER_W = ROWS // NUM_WORKERS
IDX_ROWS_PER_W = ROWS_PER_W // IDX_MINOR
GATHERS_PER_GROUP = 5
GROUP_ROWS = GATHERS_PER_GROUP * IDX_MINOR
NUM_GROUPS = ROWS_PER_W // GROUP_ROWS


def _make_kernel():
    mesh = plsc.VectorSubcoreMesh(core_axis_name="c", subcore_axis_name="s")

    @functools.partial(
        pl.kernel,
        mesh=mesh,
        out_type=jax.ShapeDtypeStruct((ROWS, DIM), jnp.float32),
        scratch_types=[
            pltpu.VMEM((IDX_ROWS_PER_W, IDX_MINOR), jnp.int32),
            pltpu.VMEM((2, GROUP_ROWS, DIM), jnp.float32),
            pltpu.SemaphoreType.DMA,
            pltpu.SemaphoreType.DMA,
        ],
        compiler_params=pltpu.CompilerParams(use_tc_tiling_on_sc=False),
    )
    def k(idx_hbm, table_hbm, out_hbm, idx_v, rows_v, sem_g, sem_s):
        wid = lax.axis_index("s") * 2 + lax.axis_index("c")
        out_base = wid * ROWS_PER_W

        pltpu.sync_copy(idx_hbm.at[wid], idx_v)

        def fire_gathers(g):
            buf = rows_v.at[g % 2]
            return [
                pltpu.async_copy(
                    table_hbm.at[idx_v.at[g * GATHERS_PER_GROUP + j]],
                    buf.at[pl.ds(j * IDX_MINOR, IDX_MINOR)],
                    sem_g,
                )
                for j in range(GATHERS_PER_GROUP)
            ]

        gather_descs = fire_gathers(0)
        store_descs = [None] * NUM_GROUPS
        for g in range(NUM_GROUPS):
            for d in gather_descs:
                d.wait()
            store_descs[g] = pltpu.async_copy(
                rows_v.at[g % 2],
                out_hbm.at[pl.ds(out_base + g * GROUP_ROWS, GROUP_ROWS)],
                sem_s,
            )
            if g + 1 < NUM_GROUPS:
                if g >= 1:
                    store_descs[g - 1].wait()
                gather_descs = fire_gathers(g + 1)
        store_descs[NUM_GROUPS - 2].wait()
        store_descs[NUM_GROUPS - 1].wait()

    return k


_gather_kernel = _make_kernel()


def kernel(indices, input_embeds):
    idx3d = indices.astype(jnp.int32).reshape(
        NUM_WORKERS, IDX_ROWS_PER_W, IDX_MINOR
    )
    out = _gather_kernel(idx3d, input_embeds)
    return out.reshape(indices.shape[0], indices.shape[1], DIM)


# speedup vs baseline: 1.0007x; 1.0007x over previous
import functools

import jax
import jax.numpy as jnp
from jax import lax
from jax.experimental import pallas as pl
from jax.experimental.pallas import tpu as pltpu
from jax.experimental.pallas import tpu_sc as plsc

VOCAB = 100000
DIM = 64
BATCH = 4096
SEQ = 50
NUM_WORKERS = 32
BATCH_PER_W = BATCH // NUM_WORKERS
GROUP = 16
NUM_GROUPS = BATCH_PER_W // GROUP


def _make_kernel():
    mesh = plsc.VectorSubcoreMesh(core_axis_name="c", subcore_axis_name="s")

    @functools.partial(
        pl.kernel,
        mesh=mesh,
        out_type=jax.ShapeDtypeStruct((BATCH, SEQ, DIM), jnp.float32),
        scratch_types=[
            pltpu.VMEM((BATCH_PER_W, SEQ), jnp.int32),
            pltpu.VMEM((GROUP, SEQ, DIM), jnp.float32),
            pltpu.SemaphoreType.DMA,
        ],
        compiler_params=pltpu.CompilerParams(use_tc_tiling_on_sc=False),
    )
    def k(idx_hbm, table_hbm, out_hbm, idx_v, rows_v, sem):
        wid = lax.axis_index("s") * 2 + lax.axis_index("c")
        base = wid * BATCH_PER_W

        pltpu.sync_copy(idx_hbm.at[pl.ds(base, BATCH_PER_W)], idx_v)

        def body(g, carry):
            descs = []
            for j in range(GROUP):
                descs.append(
                    pltpu.async_copy(
                        table_hbm.at[idx_v.at[g * GROUP + j]],
                        rows_v.at[j],
                        sem,
                    )
                )
            for d in descs:
                d.wait()
            pltpu.sync_copy(
                rows_v,
                out_hbm.at[pl.ds(base + g * GROUP, GROUP)],
            )
            return carry

        lax.fori_loop(0, NUM_GROUPS, body, 0)

    return k


_gather_kernel = _make_kernel()


def kernel(indices, input_embeds):
    out = _gather_kernel(indices.astype(jnp.int32), input_embeds)
    return out


# speedup vs baseline: 1.2259x; 1.2250x over previous
import functools

import jax
import jax.numpy as jnp
from jax import lax
from jax.experimental import pallas as pl
from jax.experimental.pallas import tpu as pltpu
from jax.experimental.pallas import tpu_sc as plsc

VOCAB = 100000
DIM = 64
PAD_DIM = 128
BATCH = 4096
SEQ = 50
NUM_WORKERS = 32
BATCH_PER_W = BATCH // NUM_WORKERS
GROUP = 8
NUM_GROUPS = BATCH_PER_W // GROUP


def _make_kernel():
    mesh = plsc.VectorSubcoreMesh(core_axis_name="c", subcore_axis_name="s")

    @functools.partial(
        pl.kernel,
        mesh=mesh,
        out_type=jax.ShapeDtypeStruct((BATCH, SEQ, PAD_DIM), jnp.float32),
        scratch_types=[
            pltpu.VMEM((BATCH_PER_W, SEQ), jnp.int32),
            pltpu.VMEM((GROUP, SEQ, PAD_DIM), jnp.float32),
            pltpu.SemaphoreType.DMA,
            pltpu.SemaphoreType.DMA,
        ],
        compiler_params=pltpu.CompilerParams(use_tc_tiling_on_sc=True),
    )
    def k(idx_hbm, table_hbm, out_hbm, idx_v, rows_v, sem_g, sem_s):
        wid = lax.axis_index("s") * 2 + lax.axis_index("c")
        base = wid * BATCH_PER_W

        pltpu.sync_copy(idx_hbm.at[pl.ds(base, BATCH_PER_W)], idx_v)

        def body(g, carry):
            descs = []
            for j in range(GROUP):
                descs.append(
                    pltpu.async_copy(
                        table_hbm.at[idx_v.at[g * GROUP + j]],
                        rows_v.at[j],
                        sem_g,
                    )
                )
            for d in descs:
                d.wait()
            pltpu.async_copy(
                rows_v,
                out_hbm.at[pl.ds(base + g * GROUP, GROUP)],
                sem_s,
            ).wait()
            return carry

        lax.fori_loop(0, NUM_GROUPS, body, 0)

    return k


_gather_kernel = _make_kernel()


def kernel(indices, input_embeds):
    table_padded = jnp.pad(input_embeds, ((0, 0), (0, PAD_DIM - DIM)))
    out = _gather_kernel(indices.astype(jnp.int32), table_padded)
    return out[:, :, :DIM]


# speedup vs baseline: 1.2747x; 1.0399x over previous
import functools

import jax
import jax.numpy as jnp
from jax import lax
from jax.experimental import pallas as pl
from jax.experimental.pallas import tpu as pltpu
from jax.experimental.pallas import tpu_sc as plsc

VOCAB = 100000
DIM = 64
PAD_DIM = 128
BATCH = 4096
SEQ = 50
NUM_WORKERS = 32
BATCH_PER_W = BATCH // NUM_WORKERS
GROUP = 8
NUM_GROUPS = BATCH_PER_W // GROUP


def _make_kernel():
    mesh = plsc.VectorSubcoreMesh(core_axis_name="c", subcore_axis_name="s")

    @functools.partial(
        pl.kernel,
        mesh=mesh,
        out_type=jax.ShapeDtypeStruct((BATCH, SEQ, PAD_DIM), jnp.float32),
        scratch_types=[
            pltpu.VMEM((BATCH_PER_W, SEQ), jnp.int32),
            pltpu.VMEM((2, GROUP, SEQ, PAD_DIM), jnp.float32),
            pltpu.SemaphoreType.DMA,
            pltpu.SemaphoreType.DMA,
        ],
        compiler_params=pltpu.CompilerParams(use_tc_tiling_on_sc=True),
    )
    def k(idx_hbm, table_hbm, out_hbm, idx_v, rows_v, sem_g, sem_s):
        wid = lax.axis_index("s") * 2 + lax.axis_index("c")
        base = wid * BATCH_PER_W

        pltpu.sync_copy(idx_hbm.at[pl.ds(base, BATCH_PER_W)], idx_v)

        def gather_descs(g):
            return [
                pltpu.make_async_copy(
                    table_hbm.at[idx_v.at[g * GROUP + j]],
                    rows_v.at[g % 2, j],
                    sem_g,
                )
                for j in range(GROUP)
            ]

        def store_desc(g):
            return pltpu.make_async_copy(
                rows_v.at[g % 2],
                out_hbm.at[pl.ds(base + g * GROUP, GROUP)],
                sem_s,
            )

        for d in gather_descs(0):
            d.start()

        def body(g, carry):
            for d in gather_descs(g):
                d.wait()

            @pl.when(g >= 1)
            def _():
                store_desc(g - 1).wait()

            @pl.when(g < NUM_GROUPS - 1)
            def _():
                for d in gather_descs(g + 1):
                    d.start()

            store_desc(g).start()
            return carry

        lax.fori_loop(0, NUM_GROUPS, body, 0)
        store_desc(NUM_GROUPS - 1).wait()

    return k


_gather_kernel = _make_kernel()


def kernel(indices, input_embeds):
    table_padded = jnp.pad(input_embeds, ((0, 0), (0, PAD_DIM - DIM)))
    out = _gather_kernel(indices.astype(jnp.int32), table_padded)
    return out[:, :, :DIM]
